# baseline (speedup 1.0000x reference)
import functools

import jax
import jax.numpy as jnp
from jax import lax
from jax.experimental import pallas as pl
from jax.experimental.pallas import tpu as pltpu
from jax.experimental.pallas import tpu_sc as plsc

VOCAB = 1000000
SEQ = 200
EMBED = 64
BATCH = 4096

_NC = 2
_NS = 16
_NW = _NC * _NS

_TOTAL_ROWS = BATCH * SEQ
_ROWS_PER_W = _TOTAL_ROWS // _NW
_IDX_BLK = 128
_BLKS_PER_CHUNK = 4
_CHUNK = _IDX_BLK * _BLKS_PER_CHUNK
_CHUNKS = _ROWS_PER_W // _CHUNK
_LANES = 16
_VPR = EMBED // _LANES


@functools.partial(
    pl.kernel,
    mesh=plsc.VectorSubcoreMesh(core_axis_name="c", subcore_axis_name="s"),
    compiler_params=pltpu.CompilerParams(use_tc_tiling_on_sc=False),
    out_type=jax.ShapeDtypeStruct((_TOTAL_ROWS, EMBED), jnp.float32),
    scratch_types=[
        pltpu.VMEM((_BLKS_PER_CHUNK, _IDX_BLK), jnp.int32),
        pltpu.VMEM((_CHUNK, EMBED), jnp.float32),
        pltpu.VMEM((SEQ, EMBED), jnp.float32),
        pltpu.SemaphoreType.DMA,
    ],
)
def _emb_kernel(idx_hbm, tok_hbm, pos_hbm, out_hbm, idx_v, rows_v, pos_v, gsem):
    wid = lax.axis_index("s") * _NC + lax.axis_index("c")
    base = wid * _ROWS_PER_W
    pltpu.sync_copy(pos_hbm, pos_v)

    def chunk_body(c, carry):
        rbase = base + c * _CHUNK
        for k in range(_BLKS_PER_CHUNK):
            pltpu.sync_copy(idx_hbm.at[pl.ds(rbase + k * _IDX_BLK, _IDX_BLK)],
                            idx_v.at[k])
        copies = []
        for k in range(_BLKS_PER_CHUNK):
            copies.append(
                pltpu.async_copy(tok_hbm.at[idx_v.at[k]],
                                 rows_v.at[pl.ds(k * _IDX_BLK, _IDX_BLK)],
                                 gsem))
        for cp in copies:
            cp.wait()

        def add_body(r, p):
            for j in range(_VPR):
                sl = pl.ds(j * _LANES, _LANES)
                rows_v[r, sl] = rows_v[r, sl] + pos_v[p, sl]
            return lax.select(p == SEQ - 1, 0, p + 1)

        lax.fori_loop(0, _CHUNK, add_body, lax.rem(c * _CHUNK, SEQ))
        pltpu.sync_copy(rows_v, out_hbm.at[pl.ds(rbase, _CHUNK)])
        return carry

    lax.fori_loop(0, _CHUNKS, chunk_body, 0)


def kernel(inputs, token_table, pos_table):
    idx = inputs.reshape(_TOTAL_ROWS)
    out = _emb_kernel(idx, token_table, pos_table)
    return out.reshape(BATCH, SEQ, EMBED)


# speedup vs baseline: 1.1269x; 1.1269x over previous
import functools

import jax
import jax.numpy as jnp
from jax import lax
from jax.experimental import pallas as pl
from jax.experimental.pallas import tpu as pltpu
from jax.experimental.pallas import tpu_sc as plsc

VOCAB = 1000000
SEQ = 200
EMBED = 64
BATCH = 4096

_NC = 2
_NS = 16
_NW = _NC * _NS

_TOTAL_ROWS = BATCH * SEQ
_ROWS_PER_W = _TOTAL_ROWS // _NW
_IDX_BLK = 128
_BLKS = 4
_CHUNK = _IDX_BLK * _BLKS
_CHUNKS = _ROWS_PER_W // _CHUNK
_LANES = 16
_VPR = EMBED // _LANES
_URF = 8


@functools.partial(
    pl.kernel,
    mesh=plsc.VectorSubcoreMesh(core_axis_name="c", subcore_axis_name="s"),
    compiler_params=pltpu.CompilerParams(use_tc_tiling_on_sc=False),
    out_type=jax.ShapeDtypeStruct((_TOTAL_ROWS, EMBED), jnp.float32),
    scratch_types=[
        pltpu.VMEM((_BLKS, _IDX_BLK), jnp.int32),
        pltpu.VMEM((_BLKS, _IDX_BLK), jnp.int32),
        pltpu.VMEM((_CHUNK, EMBED), jnp.float32),
        pltpu.VMEM((_CHUNK, EMBED), jnp.float32),
        pltpu.VMEM((SEQ, EMBED), jnp.float32),
        pltpu.SemaphoreType.DMA,
        pltpu.SemaphoreType.DMA,
        pltpu.SemaphoreType.DMA,
        pltpu.SemaphoreType.DMA,
        pltpu.SemaphoreType.DMA,
        pltpu.SemaphoreType.DMA,
    ],
)
def _emb_kernel(idx_hbm, tok_hbm, pos_hbm, out_hbm,
                idx0, idx1, rows0, rows1, pos_v,
                isem0, isem1, gsem0, gsem1, ssem0, ssem1):
    wid = lax.axis_index("s") * _NC + lax.axis_index("c")
    base = wid * _ROWS_PER_W
    pltpu.sync_copy(pos_hbm, pos_v)

    idx = (idx0, idx1)
    rows = (rows0, rows1)
    isem = (isem0, isem1)
    gsem = (gsem0, gsem1)
    ssem = (ssem0, ssem1)

    def fire_idx(c, buf):
        for k in range(_BLKS):
            pltpu.async_copy(
                idx_hbm.at[pl.ds(base + c * _CHUNK + k * _IDX_BLK, _IDX_BLK)],
                idx[buf].at[k], isem[buf])

    def wait_idx(buf):
        for k in range(_BLKS):
            pltpu.make_async_copy(idx_hbm.at[pl.ds(0, _IDX_BLK)],
                                  idx[buf].at[k], isem[buf]).wait()

    def fire_gathers(buf):
        for k in range(_BLKS):
            pltpu.async_copy(tok_hbm.at[idx[buf].at[k]],
                             rows[buf].at[pl.ds(k * _IDX_BLK, _IDX_BLK)],
                             gsem[buf])

    def wait_gathers(buf):
        for k in range(_BLKS):
            pltpu.make_async_copy(tok_hbm.at[idx[buf].at[k]],
                                  rows[buf].at[pl.ds(k * _IDX_BLK, _IDX_BLK)],
                                  gsem[buf]).wait()

    def fire_scatter(c, buf):
        pltpu.async_copy(rows[buf],
                         out_hbm.at[pl.ds(base + c * _CHUNK, _CHUNK)],
                         ssem[buf])

    def wait_scatter(buf):
        pltpu.make_async_copy(rows[buf], out_hbm.at[pl.ds(0, _CHUNK)],
                              ssem[buf]).wait()

    def add_pos(c, buf):
        r = rows[buf]

        def grp(g, p):
            for rr in range(_URF):
                row = g * _URF + rr
                for j in range(_VPR):
                    sl = pl.ds(j * _LANES, _LANES)
                    r[row, sl] = r[row, sl] + pos_v[p, sl]
                p = lax.select(p == SEQ - 1, 0, p + 1)
            return p

        lax.fori_loop(0, _CHUNK // _URF, grp, lax.rem(c * _CHUNK, SEQ),
                      unroll=False)

    def step(c, buf, fire_next_idx=True, wait_sc=True):
        obuf = 1 - buf
        wait_gathers(obuf)
        wait_idx(buf)
        if wait_sc:
            wait_scatter(buf)
        fire_gathers(buf)
        if fire_next_idx:
            fire_idx(c + 1, obuf)
        add_pos(c - 1, obuf)
        fire_scatter(c - 1, obuf)

    fire_idx(0, 0)
    wait_idx(0)
    fire_idx(1, 1)
    fire_gathers(0)
    step(1, 1, wait_sc=False)

    def super_body(i, carry):
        step(2 * i, 0)
        step(2 * i + 1, 1)
        return carry

    lax.fori_loop(1, _CHUNKS // 2 - 1, super_body, 0)

    step(_CHUNKS - 2, 0)
    step(_CHUNKS - 1, 1, fire_next_idx=False)
    wait_gathers(1)
    add_pos(_CHUNKS - 1, 1)
    fire_scatter(_CHUNKS - 1, 1)
    wait_scatter(0)
    wait_scatter(1)


def kernel(inputs, token_table, pos_table):
    idx = inputs.reshape(_TOTAL_ROWS)
    out = _emb_kernel(idx, token_table, pos_table)
    return out.reshape(BATCH, SEQ, EMBED)


# speedup vs baseline: 1.4642x; 1.2993x over previous
import functools

import jax
import jax.numpy as jnp
from jax import lax
from jax.experimental import pallas as pl
from jax.experimental.pallas import tpu as pltpu
from jax.experimental.pallas import tpu_sc as plsc

VOCAB = 1000000
SEQ = 200
EMBED = 64
BATCH = 4096

_NC = 2
_NS = 16
_NW = _NC * _NS

_B_PER_W = BATCH // _NW
_CHUNK_B = 2
_CHUNK = _CHUNK_B * SEQ
_CHUNKS = _B_PER_W // _CHUNK_B
_LANES = 16
_VPR = EMBED // _LANES
_URF = 4

_PIECES = [(0, 128), (128, 72)]


@functools.partial(
    pl.kernel,
    mesh=plsc.VectorSubcoreMesh(core_axis_name="c", subcore_axis_name="s"),
    compiler_params=pltpu.CompilerParams(use_tc_tiling_on_sc=False),
    out_type=jax.ShapeDtypeStruct((BATCH, SEQ, EMBED), jnp.float32),
    scratch_types=[
        pltpu.VMEM((_CHUNK_B, SEQ), jnp.int32),
        pltpu.VMEM((_CHUNK_B, SEQ), jnp.int32),
        pltpu.VMEM((_CHUNK_B, SEQ, EMBED), jnp.float32),
        pltpu.VMEM((_CHUNK_B, SEQ, EMBED), jnp.float32),
        pltpu.VMEM((SEQ, EMBED), jnp.float32),
        pltpu.SemaphoreType.DMA,
        pltpu.SemaphoreType.DMA,
        pltpu.SemaphoreType.DMA,
        pltpu.SemaphoreType.DMA,
        pltpu.SemaphoreType.DMA,
        pltpu.SemaphoreType.DMA,
    ],
)
def _emb_kernel(idx_hbm, tok_hbm, pos_hbm, out_hbm,
                idx0, idx1, rows0, rows1, pos_v,
                isem0, isem1, gsem0, gsem1, ssem0, ssem1):
    wid = lax.axis_index("s") * _NC + lax.axis_index("c")
    bbase = wid * _B_PER_W
    pltpu.sync_copy(pos_hbm, pos_v)

    idx = (idx0, idx1)
    rows = (rows0, rows1)
    isem = (isem0, isem1)
    gsem = (gsem0, gsem1)
    ssem = (ssem0, ssem1)

    def fire_idx(c, buf):
        pltpu.async_copy(idx_hbm.at[pl.ds(bbase + c * _CHUNK_B, _CHUNK_B)],
                         idx[buf], isem[buf])

    def wait_idx(buf):
        pltpu.make_async_copy(idx_hbm.at[pl.ds(0, _CHUNK_B)],
                              idx[buf], isem[buf]).wait()

    def fire_gathers(buf):
        for t in range(_CHUNK_B):
            for o, n in _PIECES:
                pltpu.async_copy(tok_hbm.at[idx[buf].at[t, pl.ds(o, n)]],
                                 rows[buf].at[t, pl.ds(o, n)],
                                 gsem[buf])

    def wait_gathers(buf):
        for t in range(_CHUNK_B):
            for o, n in _PIECES:
                pltpu.make_async_copy(tok_hbm.at[idx[buf].at[t, pl.ds(o, n)]],
                                      rows[buf].at[t, pl.ds(o, n)],
                                      gsem[buf]).wait()

    def fire_scatter(c, buf):
        pltpu.async_copy(rows[buf],
                         out_hbm.at[pl.ds(bbase + c * _CHUNK_B, _CHUNK_B)],
                         ssem[buf])

    def wait_scatter(buf):
        pltpu.make_async_copy(rows[buf], out_hbm.at[pl.ds(0, _CHUNK_B)],
                              ssem[buf]).wait()

    def add_pos(buf):
        r = rows[buf]

        def grp(g, carry):
            for ss in range(_URF):
                s = g * _URF + ss
                for j in range(_VPR):
                    sl = pl.ds(j * _LANES, _LANES)
                    pv = pos_v[s, sl]
                    for t in range(_CHUNK_B):
                        r[t, s, sl] = r[t, s, sl] + pv
            return carry

        lax.fori_loop(0, SEQ // _URF, grp, 0)

    def step(c, buf, fire_next_idx=True, wait_sc=True):
        obuf = 1 - buf
        wait_gathers(obuf)
        wait_idx(buf)
        if wait_sc:
            wait_scatter(buf)
        fire_gathers(buf)
        if fire_next_idx:
            fire_idx(c + 1, obuf)
        add_pos(obuf)
        fire_scatter(c - 1, obuf)

    fire_idx(0, 0)
    wait_idx(0)
    fire_idx(1, 1)
    fire_gathers(0)
    step(1, 1, wait_sc=False)

    def super_body(i, carry):
        step(2 * i, 0)
        step(2 * i + 1, 1)
        return carry

    lax.fori_loop(1, _CHUNKS // 2 - 1, super_body, 0)

    step(_CHUNKS - 2, 0)
    step(_CHUNKS - 1, 1, fire_next_idx=False)
    wait_gathers(1)
    add_pos(1)
    fire_scatter(_CHUNKS - 1, 1)
    wait_scatter(0)
    wait_scatter(1)


def kernel(inputs, token_table, pos_table):
    return _emb_kernel(inputs, token_table, pos_table)
